# Initial kernel scaffold; baseline (speedup 1.0000x reference)
import jax
import jax.numpy as jnp
from jax.experimental import pallas as pl


def kernel(x, target):
    raise NotImplementedError("write your pallas kernel here")


# speedup vs baseline: 1.6748x; 1.6748x over previous
import math

import jax
import jax.numpy as jnp
from jax.experimental import pallas as pl
from jax.experimental.pallas import tpu as pltpu

_SMOOTHING = 0.1
_CONF = 1.0 - _SMOOTHING
_RB = 512
_CB = 2048


def _loss_body(V, eps, c_row, t_ref, x_ref, out_ref):
    j = pl.program_id(1)
    xb = x_ref[...]
    t = t_ref[...]
    pad = (t == 0)
    cols = j * _CB + jax.lax.broadcasted_iota(jnp.int32, (1, _CB), 1)
    dead = (cols == 0) | (cols >= V) | pad
    xz = jnp.where(dead, 0.0, xb)
    w = jnp.where(cols == t, _CONF, eps)
    part = -jnp.sum(w * xz)
    nonpad = jnp.sum(jnp.where(pad, 0.0, c_row))

    @pl.when((pl.program_id(0) == 0) & (j == 0))
    def _init():
        out_ref[0, 0] = 0.0

    out_ref[0, 0] += part + jnp.where(j == 0, nonpad, 0.0)


def kernel(x, target):
    n, V = x.shape
    eps = _SMOOTHING / (V - 2)
    c_row = _SMOOTHING * math.log(eps) + _CONF * math.log(_CONF)
    t2 = target.astype(jnp.int32).reshape(n, 1)
    gi = n // _RB
    gj = pl.cdiv(V, _CB)

    out = pl.pallas_call(
        lambda t_ref, x_ref, out_ref: _loss_body(V, eps, c_row, t_ref, x_ref, out_ref),
        grid=(gi, gj),
        in_specs=[
            pl.BlockSpec((_RB, 1), lambda i, j: (i, 0)),
            pl.BlockSpec((_RB, _CB), lambda i, j: (i, j)),
        ],
        out_specs=pl.BlockSpec((1, 1), lambda i, j: (0, 0),
                               memory_space=pltpu.SMEM),
        out_shape=jax.ShapeDtypeStruct((1, 1), jnp.float32),
        compiler_params=pltpu.CompilerParams(
            dimension_semantics=("arbitrary", "arbitrary")),
    )(t2, x)
    return out[0, 0]
